# baseline (speedup 1.0000x reference)
import functools

import jax
import jax.numpy as jnp
from jax import lax
from jax.experimental import pallas as pl
from jax.experimental.pallas import tpu as pltpu
from jax.experimental.pallas import tpu_sc as plsc

NC = 2
NS = 16
NW = NC * NS
CH = 128
SEG = 20


def _proj1_body(x_ref, w_ref, b_ref, p_ref, r_ref):
    res = jnp.dot(x_ref[...], w_ref[...], preferred_element_type=jnp.float32)
    p_ref[...] = res[:, :16]
    r_ref[...] = res[:, 16:] + b_ref[...]


def _recip_cols(degp_ref, n, width):
    deg2 = degp_ref[0:1, :n] + degp_ref[1:2, :n]
    rec = 1.0 / jnp.maximum(deg2, 1.0)
    return jax.lax.dot_general(
        rec, jnp.ones((1, width), jnp.float32),
        (((0,), (0,)), ((), ())), preferred_element_type=jnp.float32)


def _layer1_epilogue_body(aggp_ref, degp_ref, r_ref, h_ref):
    n = r_ref.shape[0]
    agg = aggp_ref[0, :n, :] + aggp_ref[1, :n, :]
    mean = agg * _recip_cols(degp_ref, n, agg.shape[1])
    h_ref[...] = jnp.maximum(mean + r_ref[...], 0.0)


def _layer2_body(aggp_ref, degp_ref, h_ref, wl_ref, wr_ref, b_ref,
                 logp_ref, out_ref):
    n = h_ref.shape[0]
    agg = aggp_ref[0, :n, :] + aggp_ref[1, :n, :]
    mean = agg * _recip_cols(degp_ref, n, agg.shape[1])
    out = (jnp.dot(mean, wl_ref[...], preferred_element_type=jnp.float32)
           + jnp.dot(h_ref[...], wr_ref[...],
                     preferred_element_type=jnp.float32)
           + b_ref[...])
    out_ref[...] = out
    shifted = out - jnp.max(out, axis=1, keepdims=True)
    logp_ref[...] = shifted - jnp.log(
        jnp.sum(jnp.exp(shifted), axis=1, keepdims=True))


def _make_segsum(n3_rows, n_chunks, with_deg):
    rows_pt = n3_rows // NS

    mesh = plsc.VectorSubcoreMesh(core_axis_name="c", subcore_axis_name="s")
    out_type = [jax.ShapeDtypeStruct((NC, n3_rows, 16), jnp.float32)]
    if with_deg:
        out_type.append(jax.ShapeDtypeStruct((NC, n3_rows), jnp.float32))
    n_segs = n_chunks // SEG
    scratch = [
        pltpu.VMEM((n_chunks, CH), jnp.int32),
        pltpu.VMEM((n_chunks, CH), jnp.int32),
        [pltpu.VMEM((SEG * CH, 16), jnp.float32) for _ in range(2)],
        pltpu.VMEM((CH,), jnp.float32),
        pltpu.VMEM((rows_pt,), jnp.float32),
        pltpu.VMEM((rows_pt, 16), jnp.float32),
        pltpu.VMEM_SHARED((n3_rows, 16), jnp.float32),
        pltpu.VMEM_SHARED((n3_rows,), jnp.float32),
        [pltpu.SemaphoreType.DMA for _ in range(2)],
        [pltpu.SemaphoreType.DMA for _ in range(2)],
        [pltpu.SemaphoreType.DMA for _ in range(2)],
    ]
    if not with_deg:
        del scratch[10]
        del scratch[7]
        del scratch[4]
        del scratch[3]

    def body(table_h, ei_h, zeros_h, zeros1_h, ones1_h, *rest):
        if with_deg:
            (agg_out, deg_out, srcv, dstv, bufs, ones1v, degv, bounce,
             agg_s, deg_s, gsem, ssem, dsem) = rest
        else:
            (agg_out, srcv, dstv, bufs, bounce,
             agg_s, gsem, ssem) = rest
        cid = lax.axis_index("c")
        sid = lax.axis_index("s")
        wid = sid * NC + cid
        t0 = sid * rows_pt
        pltpu.sync_copy(zeros_h.at[pl.ds(t0, rows_pt)], bounce)
        pltpu.sync_copy(bounce, agg_s.at[pl.ds(t0, rows_pt)])
        if with_deg:
            pltpu.sync_copy(zeros1_h.at[pl.ds(t0, rows_pt)], degv)
            pltpu.sync_copy(degv, deg_s.at[pl.ds(t0, rows_pt)])
            pltpu.sync_copy(ones1_h, ones1v)
        pltpu.sync_copy(ei_h.at[0, wid], srcv)
        pltpu.sync_copy(ei_h.at[1, wid], dstv)
        plsc.subcore_barrier()

        def fire_gathers(s, b):
            for k in range(SEG):
                pltpu.async_copy(table_h.at[srcv.at[s * SEG + k]],
                                 bufs[b].at[pl.ds(k * CH, CH)], gsem[b])

        def drain(sem, src, dst):
            for _ in range(SEG):
                pltpu.make_async_copy(src, dst, sem).wait()

        def fire_scatters(s, b):
            for k in range(SEG):
                c = s * SEG + k
                pltpu.async_copy(bufs[b].at[pl.ds(k * CH, CH)],
                                 agg_s.at[dstv.at[c]], ssem[b], add=True)
                if with_deg:
                    pltpu.async_copy(ones1v, deg_s.at[dstv.at[c]],
                                     dsem[b], add=True)

        def drain_scatters(b):
            drain(ssem[b], bufs[b].at[pl.ds(0, CH)], agg_s.at[dstv.at[0]])
            if with_deg:
                drain(dsem[b], ones1v, deg_s.at[dstv.at[0]])

        fire_gathers(0, 0)
        for s in range(n_segs):
            b = s % 2
            if s > 0:
                drain_scatters(1 - b)
            if s + 1 < n_segs:
                fire_gathers(s + 1, 1 - b)
            drain(gsem[b], table_h.at[srcv.at[0]], bufs[b].at[pl.ds(0, CH)])
            fire_scatters(s, b)
        drain_scatters((n_segs - 1) % 2)
        plsc.subcore_barrier()
        pltpu.sync_copy(agg_s.at[pl.ds(t0, rows_pt)], bounce)
        pltpu.sync_copy(bounce, agg_out.at[cid, pl.ds(t0, rows_pt)])
        if with_deg:
            pltpu.sync_copy(deg_s.at[pl.ds(t0, rows_pt)], degv)
            pltpu.sync_copy(degv, deg_out.at[cid, pl.ds(t0, rows_pt)])

    return pl.kernel(body, out_type=out_type, mesh=mesh,
                     scratch_types=scratch,
                     compiler_params=pltpu.CompilerParams(
                         use_tc_tiling_on_sc=False))


def kernel(x, edge_index, W1l, b1l, W1r, W2l, b2l, W2r):
    n, f_in = x.shape
    dim = W1l.shape[0]
    c_out = W2l.shape[0]
    e = edge_index.shape[1]
    assert dim == 16

    n3 = -(-(n + 16) // 128) * 128
    n_chunks = -(-e // (NW * CH))
    n_chunks = -(-n_chunks // SEG) * SEG
    npad = n_chunks * NW * CH - e

    ei_p = jnp.pad(edge_index, ((0, 0), (0, npad)),
                   constant_values=n).reshape(2, NW, n_chunks, CH)
    w1 = jnp.concatenate([W1l, W1r], axis=0).T
    b1 = b1l.reshape(1, dim)
    w2l_t = W2l.T
    w2r_t = W2r.T
    b2 = b2l.reshape(1, c_out)
    zeros2 = jnp.zeros((n3, 16), jnp.float32)
    zeros1 = jnp.zeros((n3,), jnp.float32)
    ones1 = jnp.ones((CH,), jnp.float32)

    p1, r1 = pl.pallas_call(
        _proj1_body,
        out_shape=(jax.ShapeDtypeStruct((n, dim), jnp.float32),
                   jax.ShapeDtypeStruct((n, dim), jnp.float32)),
    )(x, w1, b1)

    p1_pad = jnp.pad(p1, ((0, n3 - n), (0, 0)))
    segsum_deg = _make_segsum(n3, n_chunks, with_deg=True)
    aggp1, degp = segsum_deg(p1_pad, ei_p, zeros2, zeros1, ones1)

    h = pl.pallas_call(
        _layer1_epilogue_body,
        out_shape=jax.ShapeDtypeStruct((n, dim), jnp.float32),
    )(aggp1, degp, r1)

    h_pad = jnp.pad(h, ((0, n3 - n), (0, 0)))
    segsum = _make_segsum(n3, n_chunks, with_deg=False)
    (aggp2,) = segsum(h_pad, ei_p, zeros2, zeros1, ones1)

    logp, out = pl.pallas_call(
        _layer2_body,
        out_shape=(jax.ShapeDtypeStruct((n, c_out), jnp.float32),
                   jax.ShapeDtypeStruct((n, c_out), jnp.float32)),
    )(aggp2, degp, h, w2l_t, w2r_t, b2)
    return (logp, out)


# speedup vs baseline: 1.0054x; 1.0054x over previous
import functools

import jax
import jax.numpy as jnp
from jax import lax
from jax.experimental import pallas as pl
from jax.experimental.pallas import tpu as pltpu
from jax.experimental.pallas import tpu_sc as plsc

NC = 2
NS = 16
NW = NC * NS
CH = 128
SEG = 20


def _proj1_body(x_ref, w_ref, b_ref, p_ref, r_ref):
    res = jnp.dot(x_ref[...], w_ref[...], preferred_element_type=jnp.float32)
    p_ref[...] = res[:, :16]
    r_ref[...] = res[:, 16:] + b_ref[...]


def _recip_cols(degp_ref, n, width):
    deg2 = degp_ref[0:1, :n] + degp_ref[1:2, :n]
    rec = 1.0 / jnp.maximum(deg2, 1.0)
    return jax.lax.dot_general(
        rec, jnp.ones((1, width), jnp.float32),
        (((0,), (0,)), ((), ())), preferred_element_type=jnp.float32)


def _layer1_epilogue_body(aggp_ref, degp_ref, r_ref, h_ref):
    n = r_ref.shape[0]
    agg = aggp_ref[0, :n, :] + aggp_ref[1, :n, :]
    mean = agg * _recip_cols(degp_ref, n, agg.shape[1])
    h_ref[...] = jnp.maximum(mean + r_ref[...], 0.0)


def _layer2_body(aggp_ref, degp_ref, h_ref, wl_ref, wr_ref, b_ref,
                 logp_ref, out_ref):
    n = h_ref.shape[0]
    agg = aggp_ref[0, :n, :] + aggp_ref[1, :n, :]
    mean = agg * _recip_cols(degp_ref, n, agg.shape[1])
    out = (jnp.dot(mean, wl_ref[...], preferred_element_type=jnp.float32)
           + jnp.dot(h_ref[...], wr_ref[...],
                     preferred_element_type=jnp.float32)
           + b_ref[...])
    out_ref[...] = out
    shifted = out - jnp.max(out, axis=1, keepdims=True)
    logp_ref[...] = shifted - jnp.log(
        jnp.sum(jnp.exp(shifted), axis=1, keepdims=True))


def _make_segsum(n3_rows, n_chunks, n_real, n_pad_edges, with_deg):
    rows_pt = n3_rows // NS

    mesh = plsc.VectorSubcoreMesh(core_axis_name="c", subcore_axis_name="s")
    out_type = [jax.ShapeDtypeStruct((NC, n3_rows, 16), jnp.float32)]
    if with_deg:
        out_type.append(jax.ShapeDtypeStruct((NC, n3_rows), jnp.float32))
    n_segs = n_chunks // SEG
    scratch = [
        pltpu.VMEM((n_chunks, CH), jnp.int32),
        pltpu.VMEM((n_chunks, CH), jnp.int32),
        [pltpu.VMEM((SEG * CH, 16), jnp.float32) for _ in range(2)],
        pltpu.VMEM((CH,), jnp.float32),
        pltpu.VMEM((rows_pt,), jnp.float32),
        pltpu.VMEM((rows_pt, 16), jnp.float32),
        pltpu.VMEM_SHARED((n3_rows, 16), jnp.float32),
        pltpu.VMEM_SHARED((n3_rows,), jnp.float32),
        [pltpu.SemaphoreType.DMA for _ in range(2)],
        [pltpu.SemaphoreType.DMA for _ in range(2)],
        [pltpu.SemaphoreType.DMA for _ in range(2)],
    ]
    if not with_deg:
        del scratch[10]
        del scratch[7]
        del scratch[4]
        del scratch[3]

    def body(table_h, ei_h, zeros_h, zeros1_h, ones1_h, *rest):
        if with_deg:
            (agg_out, deg_out, srcv, dstv, bufs, ones1v, degv, bounce,
             agg_s, deg_s, gsem, ssem, dsem) = rest
        else:
            (agg_out, srcv, dstv, bufs, bounce,
             agg_s, gsem, ssem) = rest
        cid = lax.axis_index("c")
        sid = lax.axis_index("s")
        wid = sid * NC + cid
        t0 = sid * rows_pt
        pltpu.sync_copy(zeros_h.at[pl.ds(t0, rows_pt)], bounce)
        pltpu.sync_copy(bounce, agg_s.at[pl.ds(t0, rows_pt)])
        if with_deg:
            pltpu.sync_copy(zeros1_h.at[pl.ds(t0, rows_pt)], degv)
            pltpu.sync_copy(degv, deg_s.at[pl.ds(t0, rows_pt)])
            pltpu.sync_copy(ones1_h, ones1v)
        pltpu.sync_copy(ei_h.at[0, wid], srcv)
        pltpu.sync_copy(ei_h.at[1, wid], dstv)
        pad_rows = n_pad_edges // CH
        if pad_rows:
            patt = n_real + lax.iota(jnp.int32, 16)

            @pl.when(wid == NW - 1)
            def _():
                for r in range(n_chunks - pad_rows, n_chunks):
                    for j in range(CH // 16):
                        dstv[r, pl.ds(j * 16, 16)] = patt
        plsc.subcore_barrier()

        def fire_gathers(s, b):
            for k in range(SEG):
                pltpu.async_copy(table_h.at[srcv.at[s * SEG + k]],
                                 bufs[b].at[pl.ds(k * CH, CH)], gsem[b])

        def drain(sem, src, dst):
            for _ in range(SEG):
                pltpu.make_async_copy(src, dst, sem).wait()

        def fire_scatters(s, b):
            for k in range(SEG):
                c = s * SEG + k
                pltpu.async_copy(bufs[b].at[pl.ds(k * CH, CH)],
                                 agg_s.at[dstv.at[c]], ssem[b], add=True)
                if with_deg:
                    pltpu.async_copy(ones1v, deg_s.at[dstv.at[c]],
                                     dsem[b], add=True)

        def drain_scatters(b):
            drain(ssem[b], bufs[b].at[pl.ds(0, CH)], agg_s.at[dstv.at[0]])
            if with_deg:
                drain(dsem[b], ones1v, deg_s.at[dstv.at[0]])

        fire_gathers(0, 0)
        for s in range(n_segs):
            b = s % 2
            if s > 0:
                drain_scatters(1 - b)
            if s + 1 < n_segs:
                fire_gathers(s + 1, 1 - b)
            drain(gsem[b], table_h.at[srcv.at[0]], bufs[b].at[pl.ds(0, CH)])
            fire_scatters(s, b)
        drain_scatters((n_segs - 1) % 2)
        plsc.subcore_barrier()
        pltpu.sync_copy(agg_s.at[pl.ds(t0, rows_pt)], bounce)
        pltpu.sync_copy(bounce, agg_out.at[cid, pl.ds(t0, rows_pt)])
        if with_deg:
            pltpu.sync_copy(deg_s.at[pl.ds(t0, rows_pt)], degv)
            pltpu.sync_copy(degv, deg_out.at[cid, pl.ds(t0, rows_pt)])

    return pl.kernel(body, out_type=out_type, mesh=mesh,
                     scratch_types=scratch,
                     compiler_params=pltpu.CompilerParams(
                         use_tc_tiling_on_sc=False))


def kernel(x, edge_index, W1l, b1l, W1r, W2l, b2l, W2r):
    n, f_in = x.shape
    dim = W1l.shape[0]
    c_out = W2l.shape[0]
    e = edge_index.shape[1]
    assert dim == 16

    n3 = -(-(n + 16) // 128) * 128
    n_chunks = -(-e // (NW * CH))
    n_chunks = -(-n_chunks // SEG) * SEG
    npad = n_chunks * NW * CH - e

    ei_p = jnp.pad(edge_index, ((0, 0), (0, npad)),
                   constant_values=n).reshape(2, NW, n_chunks, CH)
    w1 = jnp.concatenate([W1l, W1r], axis=0).T
    b1 = b1l.reshape(1, dim)
    w2l_t = W2l.T
    w2r_t = W2r.T
    b2 = b2l.reshape(1, c_out)
    zeros2 = jnp.zeros((n3, 16), jnp.float32)
    zeros1 = jnp.zeros((n3,), jnp.float32)
    ones1 = jnp.ones((CH,), jnp.float32)

    p1, r1 = pl.pallas_call(
        _proj1_body,
        out_shape=(jax.ShapeDtypeStruct((n, dim), jnp.float32),
                   jax.ShapeDtypeStruct((n, dim), jnp.float32)),
    )(x, w1, b1)

    p1_pad = jnp.pad(p1, ((0, n3 - n), (0, 0)))
    segsum_deg = _make_segsum(n3, n_chunks, n, npad, with_deg=True)
    aggp1, degp = segsum_deg(p1_pad, ei_p, zeros2, zeros1, ones1)

    h = pl.pallas_call(
        _layer1_epilogue_body,
        out_shape=jax.ShapeDtypeStruct((n, dim), jnp.float32),
    )(aggp1, degp, r1)

    h_pad = jnp.pad(h, ((0, n3 - n), (0, 0)))
    segsum = _make_segsum(n3, n_chunks, n, npad, with_deg=False)
    (aggp2,) = segsum(h_pad, ei_p, zeros2, zeros1, ones1)

    logp, out = pl.pallas_call(
        _layer2_body,
        out_shape=(jax.ShapeDtypeStruct((n, c_out), jnp.float32),
                   jax.ShapeDtypeStruct((n, c_out), jnp.float32)),
    )(aggp2, degp, h, w2l_t, w2r_t, b2)
    return (logp, out)


# speedup vs baseline: 1.5713x; 1.5629x over previous
import functools

import jax
import jax.numpy as jnp
from jax import lax
from jax.experimental import pallas as pl
from jax.experimental.pallas import tpu as pltpu
from jax.experimental.pallas import tpu_sc as plsc

NC = 2
NS = 16
NW = NC * NS
CH = 128
SEG = 20


def _proj1_body(x_ref, w_ref, b_ref, p_ref, r_ref):
    res = jnp.dot(x_ref[...], w_ref[...], preferred_element_type=jnp.float32)
    p_ref[...] = res[:, :16]
    r_ref[...] = res[:, 16:] + b_ref[...]


def _recip_cols(degp_ref, n, width):
    deg2 = degp_ref[0:1, :n] + degp_ref[1:2, :n]
    rec = 1.0 / jnp.maximum(deg2, 1.0)
    return jax.lax.dot_general(
        rec, jnp.ones((1, width), jnp.float32),
        (((0,), (0,)), ((), ())), preferred_element_type=jnp.float32)


def _layer1_epilogue_body(aggp_ref, degp_ref, r_ref, h_ref):
    n = r_ref.shape[0]
    agg = aggp_ref[0, :n, :] + aggp_ref[1, :n, :]
    mean = agg * _recip_cols(degp_ref, n, agg.shape[1])
    h_ref[...] = jnp.maximum(mean + r_ref[...], 0.0)


def _layer2_body(aggp_ref, degp_ref, h_ref, wl_ref, wr_ref, b_ref,
                 logp_ref, out_ref):
    n = h_ref.shape[0]
    agg = aggp_ref[0, :n, :] + aggp_ref[1, :n, :]
    mean = agg * _recip_cols(degp_ref, n, agg.shape[1])
    out = (jnp.dot(mean, wl_ref[...], preferred_element_type=jnp.float32)
           + jnp.dot(h_ref[...], wr_ref[...],
                     preferred_element_type=jnp.float32)
           + b_ref[...])
    out_ref[...] = out
    shifted = out - jnp.max(out, axis=1, keepdims=True)
    logp_ref[...] = shifted - jnp.log(
        jnp.sum(jnp.exp(shifted), axis=1, keepdims=True))


def _make_segsum(n3_rows, n_chunks, n_real, n_pad_edges, with_deg):
    rows_pt = n3_rows // NS

    mesh = plsc.VectorSubcoreMesh(core_axis_name="c", subcore_axis_name="s")
    out_type = [jax.ShapeDtypeStruct((NC, n3_rows, 16), jnp.float32)]
    if with_deg:
        out_type.append(jax.ShapeDtypeStruct((NC, n3_rows), jnp.float32))
    n_segs = n_chunks // SEG
    scratch = [
        pltpu.VMEM((n_chunks, CH), jnp.int32),
        pltpu.VMEM((n_chunks, CH), jnp.int32),
        [pltpu.VMEM((SEG * CH, 16), jnp.float32) for _ in range(2)],
        pltpu.VMEM((CH,), jnp.float32),
        pltpu.VMEM((rows_pt,), jnp.float32),
        pltpu.VMEM((rows_pt, 16), jnp.float32),
        pltpu.VMEM_SHARED((n3_rows, 16), jnp.float32),
        pltpu.VMEM_SHARED((n3_rows,), jnp.float32),
        [pltpu.SemaphoreType.DMA for _ in range(2)],
        [pltpu.SemaphoreType.DMA for _ in range(2)],
        [pltpu.SemaphoreType.DMA for _ in range(2)],
    ]
    if not with_deg:
        del scratch[10]
        del scratch[7]
        del scratch[4]
        del scratch[3]

    def body(table_h, ei_h, zeros_h, zeros1_h, ones1_h, *rest):
        if with_deg:
            (agg_out, deg_out, srcv, dstv, bufs, ones1v, degv, bounce,
             agg_s, deg_s, gsem, ssem, dsem) = rest
        else:
            (agg_out, srcv, dstv, bufs, bounce,
             agg_s, gsem, ssem) = rest
        cid = lax.axis_index("c")
        sid = lax.axis_index("s")
        wid = sid * NC + cid
        t0 = sid * rows_pt
        pltpu.sync_copy(zeros_h.at[pl.ds(t0, rows_pt)], bounce)
        pltpu.sync_copy(bounce, agg_s.at[pl.ds(t0, rows_pt)])
        if with_deg:
            pltpu.sync_copy(zeros1_h.at[pl.ds(t0, rows_pt)], degv)
            pltpu.sync_copy(degv, deg_s.at[pl.ds(t0, rows_pt)])
            pltpu.sync_copy(ones1_h, ones1v)
        pltpu.sync_copy(ei_h.at[0, wid], srcv)
        pltpu.sync_copy(ei_h.at[1, wid], dstv)
        pad_rows = n_pad_edges // CH
        if pad_rows:
            iot = lax.iota(jnp.int32, 16)
            dpatt = n_real + iot

            @pl.when(wid == NW - 1)
            def _():
                for r in range(n_chunks - pad_rows, n_chunks):
                    for j in range(CH // 16):
                        dstv[r, pl.ds(j * 16, 16)] = dpatt
                        srcv[r, pl.ds(j * 16, 16)] = (
                            ((r * CH + j * 16) * 131 + iot * 7) % n_real)
        plsc.subcore_barrier()

        def fire_gathers(s, b):
            for k in range(SEG):
                pltpu.async_copy(table_h.at[srcv.at[s * SEG + k]],
                                 bufs[b].at[pl.ds(k * CH, CH)], gsem[b])

        def drain(sem, src, dst):
            for _ in range(SEG):
                pltpu.make_async_copy(src, dst, sem).wait()

        def fire_scatters(s, b):
            for k in range(SEG):
                c = s * SEG + k
                pltpu.async_copy(bufs[b].at[pl.ds(k * CH, CH)],
                                 agg_s.at[dstv.at[c]], ssem[b], add=True)
                if with_deg:
                    pltpu.async_copy(ones1v, deg_s.at[dstv.at[c]],
                                     dsem[b], add=True)

        def drain_scatters(b):
            drain(ssem[b], bufs[b].at[pl.ds(0, CH)], agg_s.at[dstv.at[0]])
            if with_deg:
                drain(dsem[b], ones1v, deg_s.at[dstv.at[0]])

        fire_gathers(0, 0)
        for s in range(n_segs):
            b = s % 2
            if s > 0:
                drain_scatters(1 - b)
            if s + 1 < n_segs:
                fire_gathers(s + 1, 1 - b)
            drain(gsem[b], table_h.at[srcv.at[0]], bufs[b].at[pl.ds(0, CH)])
            fire_scatters(s, b)
        drain_scatters((n_segs - 1) % 2)
        plsc.subcore_barrier()
        pltpu.sync_copy(agg_s.at[pl.ds(t0, rows_pt)], bounce)
        pltpu.sync_copy(bounce, agg_out.at[cid, pl.ds(t0, rows_pt)])
        if with_deg:
            pltpu.sync_copy(deg_s.at[pl.ds(t0, rows_pt)], degv)
            pltpu.sync_copy(degv, deg_out.at[cid, pl.ds(t0, rows_pt)])

    return pl.kernel(body, out_type=out_type, mesh=mesh,
                     scratch_types=scratch,
                     compiler_params=pltpu.CompilerParams(
                         use_tc_tiling_on_sc=False))


def kernel(x, edge_index, W1l, b1l, W1r, W2l, b2l, W2r):
    n, f_in = x.shape
    dim = W1l.shape[0]
    c_out = W2l.shape[0]
    e = edge_index.shape[1]
    assert dim == 16

    n3 = -(-(n + 16) // 128) * 128
    n_chunks = -(-e // (NW * CH))
    n_chunks = -(-n_chunks // SEG) * SEG
    npad = n_chunks * NW * CH - e

    ei_p = jnp.pad(edge_index, ((0, 0), (0, npad)),
                   constant_values=n).reshape(2, NW, n_chunks, CH)
    w1 = jnp.concatenate([W1l, W1r], axis=0).T
    b1 = b1l.reshape(1, dim)
    w2l_t = W2l.T
    w2r_t = W2r.T
    b2 = b2l.reshape(1, c_out)
    zeros2 = jnp.zeros((n3, 16), jnp.float32)
    zeros1 = jnp.zeros((n3,), jnp.float32)
    ones1 = jnp.ones((CH,), jnp.float32)

    p1, r1 = pl.pallas_call(
        _proj1_body,
        out_shape=(jax.ShapeDtypeStruct((n, dim), jnp.float32),
                   jax.ShapeDtypeStruct((n, dim), jnp.float32)),
    )(x, w1, b1)

    p1_pad = jnp.pad(p1, ((0, n3 - n), (0, 0)))
    segsum_deg = _make_segsum(n3, n_chunks, n, npad, with_deg=True)
    aggp1, degp = segsum_deg(p1_pad, ei_p, zeros2, zeros1, ones1)

    h = pl.pallas_call(
        _layer1_epilogue_body,
        out_shape=jax.ShapeDtypeStruct((n, dim), jnp.float32),
    )(aggp1, degp, r1)

    h_pad = jnp.pad(h, ((0, n3 - n), (0, 0)))
    segsum = _make_segsum(n3, n_chunks, n, npad, with_deg=False)
    (aggp2,) = segsum(h_pad, ei_p, zeros2, zeros1, ones1)

    logp, out = pl.pallas_call(
        _layer2_body,
        out_shape=(jax.ShapeDtypeStruct((n, c_out), jnp.float32),
                   jax.ShapeDtypeStruct((n, c_out), jnp.float32)),
    )(aggp2, degp, h, w2l_t, w2r_t, b2)
    return (logp, out)
